# baseline (speedup 1.0000x reference)
import dataclasses

import jax
import jax.numpy as jnp
from jax import lax
from jax.experimental import pallas as pl
from jax.experimental.pallas import tpu as pltpu
from jax.experimental.pallas import tpu_sc as plsc

N = 10000
E = 320000
D = 128
LAM = 0.5

NC = 2
NS = 16
NW = NC * NS
CHUNK = 128
E_PAD = 327680
EDGES_PER_TILE = E_PAD // NW
CHUNKS_PER_TILE = EDGES_PER_TILE // CHUNK
NPAD = N + 48
ZSTEP = 624
ZSPAN = 640

ROW_BLOCK = 1000


def _mlp_body(h_ref, w_ref, b_ref, z_ref):
    z = lax.dot_general(
        h_ref[...], w_ref[...], (((1,), (0,)), ((), ())),
        preferred_element_type=jnp.float32,
        precision=lax.Precision.HIGHEST,
    )
    z = jnp.maximum(z + b_ref[...], 0.0)
    nrm = jnp.sqrt(jnp.sum(z * z, axis=1, keepdims=True))
    z_ref[...] = z / jnp.maximum(nrm, 1e-12)


def _mlp(h, W, b2d):
    return pl.pallas_call(
        _mlp_body,
        grid=(N // ROW_BLOCK,),
        in_specs=[
            pl.BlockSpec((ROW_BLOCK, D), lambda i: (i, 0)),
            pl.BlockSpec((D, D), lambda i: (0, 0)),
            pl.BlockSpec((1, D), lambda i: (0, 0)),
        ],
        out_specs=pl.BlockSpec((ROW_BLOCK, D), lambda i: (i, 0)),
        out_shape=jax.ShapeDtypeStruct((N, D), jnp.float32),
    )(h, W, b2d)


CHG = 1
GW = CHG * CHUNK
NG = CHUNKS_PER_TILE // CHG


def _make_hop(with_deg):
    mesh = plsc.VectorSubcoreMesh(core_axis_name="c", subcore_axis_name="s")

    out_type = [jax.ShapeDtypeStruct((NC, N, D), jnp.float32)]
    scratch = [
        pltpu.VMEM((2, CHUNK), jnp.int32),
        pltpu.VMEM((2, CHUNK), jnp.int32),
        pltpu.VMEM((2, CHUNK), jnp.int32),
        pltpu.VMEM((2, CHUNK), jnp.int32),
        pltpu.VMEM((CHUNK, D), jnp.float32),
        pltpu.VMEM((CHUNK, D), jnp.float32),
        pltpu.VMEM_SHARED((NPAD, D), jnp.float32),
    ]
    if with_deg:
        out_type.append(jax.ShapeDtypeStruct((NW, 8, NPAD), jnp.float32))
        scratch.append(pltpu.VMEM((NPAD,), jnp.float32))
    scratch += [pltpu.SemaphoreType.DMA] * 7

    def inner(table, sd3, zrows, out, degout, refs):
        (i0, i1, i2, i3, r0, r1, acc, degt,
         si0, si1, si2, si3, sg, ss0, ss1) = refs
        idx = [i0, i1, i2, i3]
        rows = [r0, r1]
        sem_i = [si0, si1, si2, si3]
        sem_s = [ss0, ss1]

        c = lax.axis_index("c")
        s = lax.axis_index("s")
        w = c * NS + s
        row0 = pl.multiple_of(s * ZSTEP, 8)
        gbase = w * CHUNKS_PER_TILE
        pltpu.sync_copy(zrows, acc.at[pl.ds(row0, ZSPAN)])
        if with_deg:
            @pl.loop(0, NPAD // 16)
            def _(i):
                degt[pl.ds(pl.multiple_of(i * 16, 16), 16)] = jnp.zeros(
                    (16,), jnp.float32)
        for k in (0, 1):
            pltpu.async_copy(sd3.at[gbase + k], idx[k], sem_i[k])
        plsc.subcore_barrier()

        @pl.loop(0, CHUNKS_PER_TILE, step=4)
        def _(g):
            for k in range(4):
                gk = g + k
                rb = k % 2
                sl = k % 4

                @pl.when(gk >= 2)
                def _():
                    pltpu.make_async_copy(table.at[pl.ds(0, CHUNK)],
                                          rows[rb], sem_s[rb]).wait()

                pltpu.make_async_copy(sd3.at[0], idx[sl], sem_i[sl]).wait()

                h = pltpu.async_copy(table.at[idx[sl].at[0]], rows[rb], sg)

                @pl.when(gk + 2 < CHUNKS_PER_TILE)
                def _():
                    pltpu.async_copy(sd3.at[gbase + gk + 2],
                                     idx[(sl + 2) % 4], sem_i[(sl + 2) % 4])

                if with_deg:
                    for t in range(CHUNK // 16):
                        iv = idx[sl][1, pl.ds(t * 16, 16)]
                        plsc.addupdate_scatter(degt, [iv],
                                               jnp.ones((16,), jnp.float32))

                h.wait()
                pltpu.async_copy(rows[rb], acc.at[idx[sl].at[1]],
                                 sem_s[rb], add=True)

        pltpu.make_async_copy(table.at[pl.ds(0, CHUNK)], rows[0],
                              sem_s[0]).wait()
        pltpu.make_async_copy(table.at[pl.ds(0, CHUNK)], rows[1],
                              sem_s[1]).wait()
        plsc.subcore_barrier()
        pltpu.sync_copy(acc.at[pl.ds(row0, ZSPAN)],
                        out.at[c, pl.ds(row0, ZSPAN)])
        if with_deg:
            pltpu.sync_copy(degt, degout.at[w, 0])

    if with_deg:
        def body(table, sd3, zrows, out, degout, *refs):
            inner(table, sd3, zrows, out, degout, refs)
    else:
        def body(table, sd3, zrows, out, *refs):
            refs = refs[:7] + (None,) + refs[7:]
            inner(table, sd3, zrows, out, None, refs)

    cp = pltpu.CompilerParams()
    if "needs_layout_passes" in pltpu.CompilerParams.__dataclass_fields__:
        cp = dataclasses.replace(cp, needs_layout_passes=False)
    return pl.kernel(body, out_type=out_type, mesh=mesh,
                     scratch_types=scratch, compiler_params=cp)


_hop_deg = _make_hop(True)
_hop = _make_hop(False)


def _c1_body(p_ref, pd_ref, out_ref):
    s = p_ref[0] + p_ref[1]
    deg = jnp.sum(pd_ref[:, 0, :], axis=0)[:N]
    out_ref[...] = s / jnp.maximum(deg, 1.0)[:, None]


def _combine1(p, pdeg):
    return pl.pallas_call(
        _c1_body,
        grid=(1,),
        in_specs=[
            pl.BlockSpec((NC, N, D), lambda i: (0, 0, 0)),
            pl.BlockSpec((NW, 8, NPAD), lambda i: (0, 0, 0)),
        ],
        out_specs=pl.BlockSpec((N, D), lambda i: (0, 0)),
        out_shape=jax.ShapeDtypeStruct((N, D), jnp.float32),
    )(p, pdeg)


def _c2_body(n1_ref, p_ref, pd_ref, out_ref):
    s = p_ref[0] + p_ref[1]
    deg = jnp.sum(pd_ref[:, 0, :], axis=0)[:N]
    neigh2 = s / jnp.maximum(deg, 1.0)[:, None]
    out_ref[...] = LAM * n1_ref[...] + (1.0 - LAM) * neigh2


def _combine2(n1, p, pdeg):
    return pl.pallas_call(
        _c2_body,
        grid=(1,),
        in_specs=[
            pl.BlockSpec((N, D), lambda i: (0, 0)),
            pl.BlockSpec((NC, N, D), lambda i: (0, 0, 0)),
            pl.BlockSpec((NW, 8, NPAD), lambda i: (0, 0, 0)),
        ],
        out_specs=pl.BlockSpec((N, D), lambda i: (0, 0)),
        out_shape=jax.ShapeDtypeStruct((N, D), jnp.float32),
    )(n1, p, pdeg)


def kernel(h, edge_index, W, b):
    z = _mlp(h, W, b.reshape(1, D))
    pad_per_tile = EDGES_PER_TILE - E // NW
    pad_src = jnp.zeros((NW, pad_per_tile), jnp.int32)
    pad_dst = jnp.broadcast_to(
        N + (jnp.arange(pad_per_tile, dtype=jnp.int32) % 48),
        (NW, pad_per_tile))
    srcp = jnp.concatenate(
        [edge_index[0].reshape(NW, E // NW), pad_src], axis=1)
    dstp = jnp.concatenate(
        [edge_index[1].reshape(NW, E // NW), pad_dst], axis=1)
    sd3 = jnp.stack([srcp.reshape(E_PAD // CHUNK, CHUNK),
                     dstp.reshape(E_PAD // CHUNK, CHUNK)], axis=1)
    zrows = jnp.zeros((ZSPAN, D), jnp.float32)
    p1, pdeg = _hop_deg(z, sd3, zrows)
    neigh1 = _combine1(p1, pdeg)
    (p2,) = _hop(neigh1, sd3, zrows)
    result = _combine2(neigh1, p2, pdeg)
    return (z, result)


# speedup vs baseline: 3.9592x; 3.9592x over previous
import dataclasses

import jax
import jax.numpy as jnp
from jax import lax
from jax.experimental import pallas as pl
from jax.experimental.pallas import tpu as pltpu
from jax.experimental.pallas import tpu_sc as plsc

N = 10000
E = 320000
D = 128
LAM = 0.5

NC = 2
NS = 16
NW = NC * NS
CHUNK = 80
EDGES_PER_TILE = E // NW
CHUNKS_PER_TILE = EDGES_PER_TILE // CHUNK
CHG = 5
NGRP = CHUNKS_PER_TILE // CHG
NPAD = N
ZSTEP = 624
ZSPAN = 640

ROW_BLOCK = 1000


def _mlp_body(h_ref, w_ref, b_ref, z_ref):
    z = lax.dot_general(
        h_ref[...], w_ref[...], (((1,), (0,)), ((), ())),
        preferred_element_type=jnp.float32,
        precision=lax.Precision.HIGHEST,
    )
    z = jnp.maximum(z + b_ref[...], 0.0)
    nrm = jnp.sqrt(jnp.sum(z * z, axis=1, keepdims=True))
    z_ref[...] = z / jnp.maximum(nrm, 1e-12)


def _mlp(h, W, b2d):
    return pl.pallas_call(
        _mlp_body,
        grid=(N // ROW_BLOCK,),
        in_specs=[
            pl.BlockSpec((ROW_BLOCK, D), lambda i: (i, 0)),
            pl.BlockSpec((D, D), lambda i: (0, 0)),
            pl.BlockSpec((1, D), lambda i: (0, 0)),
        ],
        out_specs=pl.BlockSpec((ROW_BLOCK, D), lambda i: (i, 0)),
        out_shape=jax.ShapeDtypeStruct((N, D), jnp.float32),
    )(h, W, b2d)


CHG = 1
GW = CHG * CHUNK
NG = CHUNKS_PER_TILE // CHG


def _make_hop(with_deg):
    mesh = plsc.VectorSubcoreMesh(core_axis_name="c", subcore_axis_name="s")

    out_type = [jax.ShapeDtypeStruct((NC, N, D), jnp.float32)]
    scratch = [
        pltpu.VMEM((CHG, CHUNK), jnp.int32),
        pltpu.VMEM((CHG, CHUNK), jnp.int32),
        pltpu.VMEM((CHUNK, D), jnp.float32),
        pltpu.VMEM((CHUNK, D), jnp.float32),
        pltpu.VMEM_SHARED((NPAD, D), jnp.float32),
    ]
    if with_deg:
        out_type.append(jax.ShapeDtypeStruct((NW, 8, NPAD), jnp.float32))
        scratch.append(pltpu.VMEM((NPAD,), jnp.float32))
    scratch.append(pltpu.SemaphoreType.DMA)

    def inner(table, srcg, dstg, zrows, out, degout,
              slot_s, slot_d, r0, r1, acc, degt, sg):
        rows = [r0, r1]
        c = lax.axis_index("c")
        s = lax.axis_index("s")
        w = c * NS + s
        row0 = pl.multiple_of(s * ZSTEP, 8)
        gbase = w * NGRP
        pltpu.sync_copy(zrows, acc.at[pl.ds(row0, ZSPAN)])
        if with_deg:
            @pl.loop(0, NPAD // 16)
            def _(i):
                degt[pl.ds(pl.multiple_of(i * 16, 16), 16)] = jnp.zeros(
                    (16,), jnp.float32)
        plsc.subcore_barrier()

        @pl.loop(0, NGRP)
        def _(g):
            r = gbase + g
            pltpu.sync_copy(srcg.at[r], slot_s)
            pltpu.sync_copy(dstg.at[r], slot_d)
            h = pltpu.async_copy(table.at[slot_s.at[0]], rows[0], sg)
            for j in range(CHG):
                h2 = None
                if j + 1 < CHG:
                    h2 = pltpu.async_copy(table.at[slot_s.at[j + 1]],
                                          rows[(j + 1) % 2], sg)
                h.wait()
                pltpu.sync_copy(rows[j % 2], acc.at[slot_d.at[j]], add=True)
                if with_deg:
                    for t in range(CHUNK // 16):
                        iv = slot_d[j, pl.ds(t * 16, 16)]
                        plsc.addupdate_scatter(degt, [iv],
                                               jnp.ones((16,), jnp.float32))
                h = h2

        plsc.subcore_barrier()
        pltpu.sync_copy(acc.at[pl.ds(row0, ZSPAN)],
                        out.at[c, pl.ds(row0, ZSPAN)])
        if with_deg:
            pltpu.sync_copy(degt, degout.at[w, 0])

    if with_deg:
        def body(table, srcg, dstg, zrows, out, degout,
                 slot_s, slot_d, r0, r1, acc, degt, sg):
            inner(table, srcg, dstg, zrows, out, degout,
                  slot_s, slot_d, r0, r1, acc, degt, sg)
    else:
        def body(table, srcg, dstg, zrows, out,
                 slot_s, slot_d, r0, r1, acc, sg):
            inner(table, srcg, dstg, zrows, out, None,
                  slot_s, slot_d, r0, r1, acc, None, sg)

    cp = pltpu.CompilerParams()
    if "needs_layout_passes" in pltpu.CompilerParams.__dataclass_fields__:
        cp = dataclasses.replace(cp, needs_layout_passes=False)
    return pl.kernel(body, out_type=out_type, mesh=mesh,
                     scratch_types=scratch, compiler_params=cp)


_hop_deg = _make_hop(True)
_hop = _make_hop(False)


def _c1_body(p_ref, pd_ref, out_ref):
    s = p_ref[0] + p_ref[1]
    deg = jnp.sum(pd_ref[:, 0, :], axis=0)[:N]
    out_ref[...] = s / jnp.maximum(deg, 1.0)[:, None]


def _combine1(p, pdeg):
    return pl.pallas_call(
        _c1_body,
        grid=(1,),
        in_specs=[
            pl.BlockSpec((NC, N, D), lambda i: (0, 0, 0)),
            pl.BlockSpec((NW, 8, NPAD), lambda i: (0, 0, 0)),
        ],
        out_specs=pl.BlockSpec((N, D), lambda i: (0, 0)),
        out_shape=jax.ShapeDtypeStruct((N, D), jnp.float32),
    )(p, pdeg)


def _c2_body(n1_ref, p_ref, pd_ref, out_ref):
    s = p_ref[0] + p_ref[1]
    deg = jnp.sum(pd_ref[:, 0, :], axis=0)[:N]
    neigh2 = s / jnp.maximum(deg, 1.0)[:, None]
    out_ref[...] = LAM * n1_ref[...] + (1.0 - LAM) * neigh2


def _combine2(n1, p, pdeg):
    return pl.pallas_call(
        _c2_body,
        grid=(1,),
        in_specs=[
            pl.BlockSpec((N, D), lambda i: (0, 0)),
            pl.BlockSpec((NC, N, D), lambda i: (0, 0, 0)),
            pl.BlockSpec((NW, 8, NPAD), lambda i: (0, 0, 0)),
        ],
        out_specs=pl.BlockSpec((N, D), lambda i: (0, 0)),
        out_shape=jax.ShapeDtypeStruct((N, D), jnp.float32),
    )(n1, p, pdeg)


def kernel(h, edge_index, W, b):
    z = _mlp(h, W, b.reshape(1, D))
    srcg = edge_index[0].reshape(E // (CHG * CHUNK), CHG, CHUNK)
    dstg = edge_index[1].reshape(E // (CHG * CHUNK), CHG, CHUNK)
    zrows = jnp.zeros((ZSPAN, D), jnp.float32)
    p1, pdeg = _hop_deg(z, srcg, dstg, zrows)
    neigh1 = _combine1(p1, pdeg)
    (p2,) = _hop(neigh1, srcg, dstg, zrows)
    result = _combine2(neigh1, p2, pdeg)
    return (z, result)


# speedup vs baseline: 3.9614x; 1.0006x over previous
import dataclasses

import jax
import jax.numpy as jnp
from jax import lax
from jax.experimental import pallas as pl
from jax.experimental.pallas import tpu as pltpu
from jax.experimental.pallas import tpu_sc as plsc

N = 10000
E = 320000
D = 128
LAM = 0.5

NC = 2
NS = 16
NW = NC * NS
CHUNK = 80
EDGES_PER_TILE = E // NW
CHUNKS_PER_TILE = EDGES_PER_TILE // CHUNK
CHG = 5
NGRP = CHUNKS_PER_TILE // CHG
NPAD = N
ZSTEP = 624
ZSPAN = 640

ROW_BLOCK = 1000


def _mlp_body(h_ref, w_ref, b_ref, z_ref):
    z = lax.dot_general(
        h_ref[...], w_ref[...], (((1,), (0,)), ((), ())),
        preferred_element_type=jnp.float32,
        precision=lax.Precision.HIGHEST,
    )
    z = jnp.maximum(z + b_ref[...], 0.0)
    nrm = jnp.sqrt(jnp.sum(z * z, axis=1, keepdims=True))
    z_ref[...] = z / jnp.maximum(nrm, 1e-12)


def _mlp(h, W, b2d):
    return pl.pallas_call(
        _mlp_body,
        grid=(N // ROW_BLOCK,),
        in_specs=[
            pl.BlockSpec((ROW_BLOCK, D), lambda i: (i, 0)),
            pl.BlockSpec((D, D), lambda i: (0, 0)),
            pl.BlockSpec((1, D), lambda i: (0, 0)),
        ],
        out_specs=pl.BlockSpec((ROW_BLOCK, D), lambda i: (i, 0)),
        out_shape=jax.ShapeDtypeStruct((N, D), jnp.float32),
    )(h, W, b2d)


CHG = 1
GW = CHG * CHUNK
NG = CHUNKS_PER_TILE // CHG


def _make_hop(with_deg):
    mesh = plsc.VectorSubcoreMesh(core_axis_name="c", subcore_axis_name="s")

    out_type = [jax.ShapeDtypeStruct((NC, N, D), jnp.float32)]
    scratch = [
        pltpu.VMEM((CHG, CHUNK), jnp.int32),
        pltpu.VMEM((CHG, CHUNK), jnp.int32),
        pltpu.VMEM((CHUNK, D), jnp.float32),
        pltpu.VMEM((CHUNK, D), jnp.float32),
        pltpu.VMEM_SHARED((NPAD, D), jnp.float32),
    ]
    if with_deg:
        out_type.append(jax.ShapeDtypeStruct((NW, 8, NPAD), jnp.float32))
        scratch.append(pltpu.VMEM((NPAD,), jnp.float32))
    scratch += [pltpu.SemaphoreType.DMA] * 2

    def inner(table, srcg, dstg, zrows, out, degout,
              slot_s, slot_d, r0, r1, acc, degt, sg0, sg1):
        rows = [r0, r1]
        sg = [sg0, sg1]
        c = lax.axis_index("c")
        s = lax.axis_index("s")
        w = c * NS + s
        row0 = pl.multiple_of(s * ZSTEP, 8)
        gbase = w * NGRP
        pltpu.sync_copy(zrows, acc.at[pl.ds(row0, ZSPAN)])
        if with_deg:
            @pl.loop(0, NPAD // 16)
            def _(i):
                degt[pl.ds(pl.multiple_of(i * 16, 16), 16)] = jnp.zeros(
                    (16,), jnp.float32)
        plsc.subcore_barrier()

        @pl.loop(0, NGRP)
        def _(g):
            r = gbase + g
            pltpu.sync_copy(srcg.at[r], slot_s)
            pltpu.sync_copy(dstg.at[r], slot_d)
            h = pltpu.async_copy(table.at[slot_s.at[0]], rows[0], sg[0])
            for j in range(CHG):
                h2 = None
                if j + 1 < CHG:
                    h2 = pltpu.async_copy(table.at[slot_s.at[j + 1]],
                                          rows[(j + 1) % 2], sg[(j + 1) % 2])
                h.wait()
                pltpu.sync_copy(rows[j % 2], acc.at[slot_d.at[j]], add=True)
                if with_deg:
                    for t in range(CHUNK // 16):
                        iv = slot_d[j, pl.ds(t * 16, 16)]
                        plsc.addupdate_scatter(degt, [iv],
                                               jnp.ones((16,), jnp.float32))
                h = h2

        plsc.subcore_barrier()
        pltpu.sync_copy(acc.at[pl.ds(row0, ZSPAN)],
                        out.at[c, pl.ds(row0, ZSPAN)])
        if with_deg:
            pltpu.sync_copy(degt, degout.at[w, 0])

    if with_deg:
        def body(table, srcg, dstg, zrows, out, degout,
                 slot_s, slot_d, r0, r1, acc, degt, sg0, sg1):
            inner(table, srcg, dstg, zrows, out, degout,
                  slot_s, slot_d, r0, r1, acc, degt, sg0, sg1)
    else:
        def body(table, srcg, dstg, zrows, out,
                 slot_s, slot_d, r0, r1, acc, sg0, sg1):
            inner(table, srcg, dstg, zrows, out, None,
                  slot_s, slot_d, r0, r1, acc, None, sg0, sg1)

    cp = pltpu.CompilerParams()
    if "needs_layout_passes" in pltpu.CompilerParams.__dataclass_fields__:
        cp = dataclasses.replace(cp, needs_layout_passes=False)
    return pl.kernel(body, out_type=out_type, mesh=mesh,
                     scratch_types=scratch, compiler_params=cp)


_hop_deg = _make_hop(True)
_hop = _make_hop(False)


def _c1_body(p_ref, pd_ref, out_ref):
    s = p_ref[0] + p_ref[1]
    deg = jnp.sum(pd_ref[:, 0, :], axis=0)[:N]
    out_ref[...] = s / jnp.maximum(deg, 1.0)[:, None]


def _combine1(p, pdeg):
    return pl.pallas_call(
        _c1_body,
        grid=(1,),
        in_specs=[
            pl.BlockSpec((NC, N, D), lambda i: (0, 0, 0)),
            pl.BlockSpec((NW, 8, NPAD), lambda i: (0, 0, 0)),
        ],
        out_specs=pl.BlockSpec((N, D), lambda i: (0, 0)),
        out_shape=jax.ShapeDtypeStruct((N, D), jnp.float32),
    )(p, pdeg)


def _c2_body(n1_ref, p_ref, pd_ref, out_ref):
    s = p_ref[0] + p_ref[1]
    deg = jnp.sum(pd_ref[:, 0, :], axis=0)[:N]
    neigh2 = s / jnp.maximum(deg, 1.0)[:, None]
    out_ref[...] = LAM * n1_ref[...] + (1.0 - LAM) * neigh2


def _combine2(n1, p, pdeg):
    return pl.pallas_call(
        _c2_body,
        grid=(1,),
        in_specs=[
            pl.BlockSpec((N, D), lambda i: (0, 0)),
            pl.BlockSpec((NC, N, D), lambda i: (0, 0, 0)),
            pl.BlockSpec((NW, 8, NPAD), lambda i: (0, 0, 0)),
        ],
        out_specs=pl.BlockSpec((N, D), lambda i: (0, 0)),
        out_shape=jax.ShapeDtypeStruct((N, D), jnp.float32),
    )(n1, p, pdeg)


def kernel(h, edge_index, W, b):
    z = _mlp(h, W, b.reshape(1, D))
    srcg = edge_index[0].reshape(E // (CHG * CHUNK), CHG, CHUNK)
    dstg = edge_index[1].reshape(E // (CHG * CHUNK), CHG, CHUNK)
    zrows = jnp.zeros((ZSPAN, D), jnp.float32)
    p1, pdeg = _hop_deg(z, srcg, dstg, zrows)
    neigh1 = _combine1(p1, pdeg)
    (p2,) = _hop(neigh1, srcg, dstg, zrows)
    result = _combine2(neigh1, p2, pdeg)
    return (z, result)
